# baseline (speedup 1.0000x reference)
import functools

import jax
import jax.numpy as jnp
from jax.experimental import pallas as pl
from jax.experimental.pallas import tpu as pltpu

_TOP_K = 30
_BS, _H, _W = 2, 32, 32
_HW = _H * _W
_NM = 8192
_CK = 64
_CV = 256
_CP = 1024
_NOBJ = 16
_TILE = 512
_NTILES = _HW // _TILE
_NSLC = 8
_SLC = _NM // _NSLC
_BISECT_ITERS = 13


def _dot(a, b, dims, out_dtype=jnp.float32):
    return jax.lax.dot_general(a, b, (dims, ((), ())),
                               preferred_element_type=out_dtype)


def _body(lhs_ref, rhs_ref, v_ref, pf_ref, lpf_ref, lmv_ref, sens_ref, lm_ref,
          obj_ref, wu1lpf_ref, wu1pf_ref, wu1lm_ref, wu1d_ref, bu1_ref,
          wu2_ref, bu2_ref, wfpf_ref, wfvr_ref, wfs_ref, wflm_ref, bf_ref,
          wq_ref, wk_ref, wv_ref, wo_ref, out_ref):
    f32 = jnp.float32
    lhs = lhs_ref[0]
    rhs = rhs_ref[0]
    sim = _dot(lhs, rhs, ((1,), (0,)))

    _RB = 64
    blocks = []
    for rb in range(0, _TILE, _RB):
        s = sim[rb:rb + _RB]
        m1 = s[:, 0:_SLC]
        m2 = jnp.full_like(m1, -1e30)
        for k in range(1, _NSLC):
            x = s[:, k * _SLC:(k + 1) * _SLC]
            m2 = jnp.maximum(m2, jnp.minimum(m1, x))
            m1 = jnp.maximum(m1, x)
        u1 = jnp.full((_RB, _SLC // 16), -1e30, f32)
        u2, u3, u4 = u1, u1, u1
        for srcarr in (m1, m2):
            for k in range(16):
                x = srcarr[:, k * (_SLC // 16):(k + 1) * (_SLC // 16)]
                r = jnp.minimum(u1, x)
                u1 = jnp.maximum(u1, x)
                r2 = jnp.minimum(u2, r)
                u2 = jnp.maximum(u2, r)
                r3 = jnp.minimum(u3, r2)
                u3 = jnp.maximum(u3, r2)
                u4 = jnp.maximum(u4, r3)
        blk = jnp.concatenate([u1, u2, u3, u4], axis=1)
        blocks.append(jnp.swapaxes(blk, 0, 1))
    cand_t = jnp.concatenate(blocks, axis=1)
    row_max_t = jnp.max(cand_t[0:_SLC // 16], axis=0, keepdims=True)
    lo0_t = jnp.min(cand_t[0:_SLC // 16], axis=0, keepdims=True)

    def bis(_, c):
        lo, hi = c
        mid = 0.5 * (lo + hi)
        cnt = jnp.sum((cand_t >= mid).astype(f32), axis=0, keepdims=True)
        ge = cnt >= float(_TOP_K)
        return jnp.where(ge, mid, lo), jnp.where(ge, hi, mid)

    thr_t, _ = jax.lax.fori_loop(0, _BISECT_ITERS, bis, (lo0_t, row_max_t))
    thr = jnp.swapaxes(thr_t, 0, 1)
    row_max = jnp.swapaxes(row_max_t, 0, 1)

    pb = jnp.where(sim >= thr, jnp.exp(sim - row_max),
                   0.0).astype(jnp.bfloat16)
    r = _dot(v_ref[0], pb, ((1,), (1,)))
    vr = r[:_CV] * (1.0 / r[_CV:_CV + 1])

    pf = pf_ref[0]
    lpf = lpf_ref[0]
    lmv = lmv_ref[0]
    lm = lm_ref[0]
    diff = vr - lmv
    h1 = (_dot(wu1lpf_ref[...], lpf, ((1,), (0,)))
          + _dot(wu1pf_ref[...], pf, ((1,), (0,)))
          + wu1lm_ref[...] * lm
          + _dot(wu1d_ref[...], diff, ((1,), (0,)))
          + bu1_ref[...])
    h1 = jnp.maximum(h1, 0.0)
    logits = _dot(wu2_ref[...], h1, ((1,), (0,))) + bu2_ref[...]
    up = jax.nn.sigmoid(logits)
    vr2 = vr * up + lmv * (1.0 - up)

    prd = (_dot(wfpf_ref[...], pf, ((1,), (0,)))
           + _dot(wfvr_ref[...], vr2, ((1,), (0,)))
           + _dot(wfs_ref[...], sens_ref[0], ((1,), (0,)))
           + wflm_ref[...] * lm
           + bf_ref[...])
    prd = jnp.maximum(prd, 0.0)

    obj = obj_ref[0]
    k_ = _dot(obj, wk_ref[...], ((1,), (1,)))
    v_ = _dot(obj, wv_ref[...], ((1,), (1,)))
    q_t = _dot(wq_ref[...], prd, ((1,), (0,)))
    al = _dot(k_, q_t, ((1,), (0,))) * (1.0 / 16.0)
    al = al - jnp.max(al, axis=0, keepdims=True)
    ae = jnp.exp(al)
    attn = ae / jnp.sum(ae, axis=0, keepdims=True)
    o_t = _dot(v_, attn, ((0,), (0,)))
    out_ref[0, 0] = prd + _dot(wo_ref[...], o_t, ((1,), (0,)))


@functools.partial(jax.jit, static_argnums=())
def kernel(query_key, query_selection, pix_feat, sensory, last_mask,
           last_pix_feat, last_msk_value, mem_key, mem_shrinkage,
           mem_msk_value, mem_valid, obj_memory, W_u1, b_u1, W_u2, b_u2,
           W_f, b_f, Wq, Wk, Wv, Wo):
    f32 = jnp.float32
    w = mem_shrinkage[:, 0, :] * (1.0 / (_CK ** 0.5))
    neginv = (1.0 - mem_valid) * (-60000.0)
    mk = mem_key
    rhs = jnp.concatenate(
        [mk * mk * w[:, None, :], mk * w[:, None, :], -w[:, None, :],
         neginv[:, None, :]], axis=1)

    qk = query_key.reshape(_BS, _CK, _HW)
    qe = query_selection.reshape(_BS, _CK, _HW)
    qe_t = qe.transpose(0, 2, 1)
    qq_t = (qk * qe).transpose(0, 2, 1)
    bsq = jnp.sum(qe * qk * qk, axis=1)
    ones = jnp.ones((_BS, _HW, 1), f32)
    lhs = jnp.concatenate([-qe_t, 2.0 * qq_t, bsq[..., None], ones],
                          axis=-1)

    v16 = jnp.concatenate(
        [mem_msk_value, jnp.ones((_BS, 1, _NM), f32)],
        axis=1).astype(jnp.bfloat16)
    pf = pix_feat.reshape(_BS, _CP, _HW)
    lpf = last_pix_feat.reshape(_BS, _CP, _HW)
    lmv = last_msk_value.reshape(_BS, _CV, _HW)
    sens = sensory.reshape(_BS, _CV, _HW)
    lm = last_mask.reshape(_BS, 1, _HW)
    obj = obj_memory[:, 0]

    wu1lpf = W_u1[:, :_CP]
    wu1pf = W_u1[:, _CP:2 * _CP]
    wu1lm = W_u1[:, 2 * _CP:2 * _CP + 1]
    wu1d = W_u1[:, 2 * _CP + 1:]
    wfpf = W_f[:, :_CP]
    wfvr = W_f[:, _CP:_CP + _CV]
    wfs = W_f[:, _CP + _CV:_CP + 2 * _CV]
    wflm = W_f[:, _CP + 2 * _CV:]
    bu1 = b_u1[:, None]
    bu2 = b_u2[:, None]
    bf_ = b_f[:, None]

    grid = (_BS, _NTILES)
    bspec = pl.BlockSpec
    full = lambda shape: bspec(shape, lambda b, i: (0,) * len(shape))
    out = pl.pallas_call(
        _body,
        grid=grid,
        in_specs=[
            bspec((1, _TILE, 130), lambda b, i: (b, i, 0)),
            bspec((1, 130, _NM), lambda b, i: (b, 0, 0)),
            bspec((1, _CV + 1, _NM), lambda b, i: (b, 0, 0)),
            bspec((1, _CP, _TILE), lambda b, i: (b, 0, i)),
            bspec((1, _CP, _TILE), lambda b, i: (b, 0, i)),
            bspec((1, _CV, _TILE), lambda b, i: (b, 0, i)),
            bspec((1, _CV, _TILE), lambda b, i: (b, 0, i)),
            bspec((1, 1, _TILE), lambda b, i: (b, 0, i)),
            bspec((1, _NOBJ, _CV), lambda b, i: (b, 0, 0)),
            full((64, _CP)), full((64, _CP)), full((64, 1)),
            full((64, _CV)), full((64, 1)),
            full((1, 64)), full((1, 1)),
            full((_CV, _CP)), full((_CV, _CV)), full((_CV, _CV)),
            full((_CV, 1)), full((_CV, 1)),
            full((_CV, _CV)), full((_CV, _CV)), full((_CV, _CV)),
            full((_CV, _CV)),
        ],
        out_specs=bspec((1, 1, _CV, _TILE), lambda b, i: (b, 0, 0, i)),
        out_shape=jax.ShapeDtypeStruct((_BS, 1, _CV, _HW), f32),
        compiler_params=pltpu.CompilerParams(
            dimension_semantics=("parallel", "parallel"),
        ),
    )(lhs, rhs, v16, pf, lpf, lmv, sens, lm, obj,
      wu1lpf, wu1pf, wu1lm, wu1d, bu1, W_u2, bu2,
      wfpf, wfvr, wfs, wflm, bf_, Wq, Wk, Wv, Wo)
    return out.reshape(_BS, 1, _CV, _H, _W)


# speedup vs baseline: 1.0389x; 1.0389x over previous
import functools

import jax
import jax.numpy as jnp
from jax.experimental import pallas as pl
from jax.experimental.pallas import tpu as pltpu

_TOP_K = 30
_BS, _H, _W = 2, 32, 32
_HW = _H * _W
_NM = 8192
_CK = 64
_CV = 256
_CP = 1024
_NOBJ = 16
_TILE = 512
_NTILES = _HW // _TILE
_NSLC = 8
_SLC = _NM // _NSLC
_BISECT_ITERS = 13


def _dot(a, b, dims, out_dtype=jnp.float32):
    return jax.lax.dot_general(a, b, (dims, ((), ())),
                               preferred_element_type=out_dtype)


def _body(lhs_ref, rhs_ref, v_ref, pf_ref, lpf_ref, lmv_ref, sens_ref, lm_ref,
          obj_ref, wu1lpf_ref, wu1pf_ref, wu1lm_ref, wu1d_ref, bu1_ref,
          wu2_ref, bu2_ref, wfpf_ref, wfvr_ref, wfs_ref, wflm_ref, bf_ref,
          wq_ref, wk_ref, wv_ref, wo_ref, out_ref):
    f32 = jnp.float32
    lhs = lhs_ref[0]
    rhs = rhs_ref[0]
    sim = _dot(lhs, rhs, ((1,), (0,)))

    m1 = sim[:, 0:_SLC]
    m2 = jnp.full_like(m1, -1e30)
    for k in range(1, _NSLC):
        x = sim[:, k * _SLC:(k + 1) * _SLC]
        m2 = jnp.maximum(m2, jnp.minimum(m1, x))
        m1 = jnp.maximum(m1, x)
    u1 = jnp.full((sim.shape[0], _SLC // 16), -1e30, f32)
    u2, u3, u4 = u1, u1, u1
    for srcarr in (m1, m2):
        for k in range(16):
            x = srcarr[:, k * (_SLC // 16):(k + 1) * (_SLC // 16)]
            r = jnp.minimum(u1, x)
            u1 = jnp.maximum(u1, x)
            r2 = jnp.minimum(u2, r)
            u2 = jnp.maximum(u2, r)
            r3 = jnp.minimum(u3, r2)
            u3 = jnp.maximum(u3, r2)
            u4 = jnp.maximum(u4, r3)
    cand_t = jnp.swapaxes(
        jnp.concatenate([u1, u2, u3, u4], axis=1), 0, 1)
    row_max_t = jnp.max(cand_t[0:_SLC // 16], axis=0, keepdims=True)
    lo0_t = jnp.min(cand_t[0:_SLC // 16], axis=0, keepdims=True)

    def bis(_, c):
        lo, hi = c
        mid = 0.5 * (lo + hi)
        cnt = jnp.sum((cand_t >= mid).astype(f32), axis=0, keepdims=True)
        ge = cnt >= float(_TOP_K)
        return jnp.where(ge, mid, lo), jnp.where(ge, hi, mid)

    thr_t, _ = jax.lax.fori_loop(0, _BISECT_ITERS, bis, (lo0_t, row_max_t))
    thr = jnp.swapaxes(thr_t, 0, 1)
    row_max = jnp.swapaxes(row_max_t, 0, 1)

    pb = jnp.where(sim >= thr, jnp.exp(sim - row_max),
                   0.0).astype(jnp.bfloat16)
    r = _dot(v_ref[0], pb, ((1,), (1,)))
    vr = r[:_CV] * (1.0 / r[_CV:_CV + 1])

    pf = pf_ref[0]
    lpf = lpf_ref[0]
    lmv = lmv_ref[0]
    lm = lm_ref[0]
    diff = vr - lmv
    h1 = (_dot(wu1lpf_ref[...], lpf, ((1,), (0,)))
          + _dot(wu1pf_ref[...], pf, ((1,), (0,)))
          + wu1lm_ref[...] * lm
          + _dot(wu1d_ref[...], diff, ((1,), (0,)))
          + bu1_ref[...])
    h1 = jnp.maximum(h1, 0.0)
    logits = _dot(wu2_ref[...], h1, ((1,), (0,))) + bu2_ref[...]
    up = jax.nn.sigmoid(logits)
    vr2 = vr * up + lmv * (1.0 - up)

    prd = (_dot(wfpf_ref[...], pf, ((1,), (0,)))
           + _dot(wfvr_ref[...], vr2, ((1,), (0,)))
           + _dot(wfs_ref[...], sens_ref[0], ((1,), (0,)))
           + wflm_ref[...] * lm
           + bf_ref[...])
    prd = jnp.maximum(prd, 0.0)

    obj = obj_ref[0]
    k_ = _dot(obj, wk_ref[...], ((1,), (1,)))
    v_ = _dot(obj, wv_ref[...], ((1,), (1,)))
    q_t = _dot(wq_ref[...], prd, ((1,), (0,)))
    al = _dot(k_, q_t, ((1,), (0,))) * (1.0 / 16.0)
    al = al - jnp.max(al, axis=0, keepdims=True)
    ae = jnp.exp(al)
    attn = ae / jnp.sum(ae, axis=0, keepdims=True)
    o_t = _dot(v_, attn, ((0,), (0,)))
    out_ref[0, 0] = prd + _dot(wo_ref[...], o_t, ((1,), (0,)))


@functools.partial(jax.jit, static_argnums=())
def kernel(query_key, query_selection, pix_feat, sensory, last_mask,
           last_pix_feat, last_msk_value, mem_key, mem_shrinkage,
           mem_msk_value, mem_valid, obj_memory, W_u1, b_u1, W_u2, b_u2,
           W_f, b_f, Wq, Wk, Wv, Wo):
    f32 = jnp.float32
    w = mem_shrinkage[:, 0, :] * (1.0 / (_CK ** 0.5))
    neginv = (1.0 - mem_valid) * (-60000.0)
    mk = mem_key
    rhs = jnp.concatenate(
        [mk * mk * w[:, None, :], mk * w[:, None, :], -w[:, None, :],
         neginv[:, None, :]], axis=1)

    qk = query_key.reshape(_BS, _CK, _HW)
    qe = query_selection.reshape(_BS, _CK, _HW)
    qe_t = qe.transpose(0, 2, 1)
    qq_t = (qk * qe).transpose(0, 2, 1)
    bsq = jnp.sum(qe * qk * qk, axis=1)
    ones = jnp.ones((_BS, _HW, 1), f32)
    lhs = jnp.concatenate([-qe_t, 2.0 * qq_t, bsq[..., None], ones],
                          axis=-1)

    v16 = jnp.concatenate(
        [mem_msk_value, jnp.ones((_BS, 1, _NM), f32)],
        axis=1).astype(jnp.bfloat16)
    bf16 = jnp.bfloat16
    pf = pix_feat.reshape(_BS, _CP, _HW).astype(bf16)
    lpf = last_pix_feat.reshape(_BS, _CP, _HW).astype(bf16)
    lmv = last_msk_value.reshape(_BS, _CV, _HW)
    sens = sensory.reshape(_BS, _CV, _HW)
    lm = last_mask.reshape(_BS, 1, _HW)
    obj = obj_memory[:, 0]

    wu1lpf = W_u1[:, :_CP].astype(bf16)
    wu1pf = W_u1[:, _CP:2 * _CP].astype(bf16)
    wu1lm = W_u1[:, 2 * _CP:2 * _CP + 1]
    wu1d = W_u1[:, 2 * _CP + 1:]
    wfpf = W_f[:, :_CP].astype(bf16)
    wfvr = W_f[:, _CP:_CP + _CV]
    wfs = W_f[:, _CP + _CV:_CP + 2 * _CV]
    wflm = W_f[:, _CP + 2 * _CV:]
    bu1 = b_u1[:, None]
    bu2 = b_u2[:, None]
    bf_ = b_f[:, None]

    grid = (_BS, _NTILES)
    bspec = pl.BlockSpec
    full = lambda shape: bspec(shape, lambda b, i: (0,) * len(shape))
    out = pl.pallas_call(
        _body,
        grid=grid,
        in_specs=[
            bspec((1, _TILE, 130), lambda b, i: (b, i, 0)),
            bspec((1, 130, _NM), lambda b, i: (b, 0, 0)),
            bspec((1, _CV + 1, _NM), lambda b, i: (b, 0, 0)),
            bspec((1, _CP, _TILE), lambda b, i: (b, 0, i)),
            bspec((1, _CP, _TILE), lambda b, i: (b, 0, i)),
            bspec((1, _CV, _TILE), lambda b, i: (b, 0, i)),
            bspec((1, _CV, _TILE), lambda b, i: (b, 0, i)),
            bspec((1, 1, _TILE), lambda b, i: (b, 0, i)),
            bspec((1, _NOBJ, _CV), lambda b, i: (b, 0, 0)),
            full((64, _CP)), full((64, _CP)), full((64, 1)),
            full((64, _CV)), full((64, 1)),
            full((1, 64)), full((1, 1)),
            full((_CV, _CP)), full((_CV, _CV)), full((_CV, _CV)),
            full((_CV, 1)), full((_CV, 1)),
            full((_CV, _CV)), full((_CV, _CV)), full((_CV, _CV)),
            full((_CV, _CV)),
        ],
        out_specs=bspec((1, 1, _CV, _TILE), lambda b, i: (b, 0, 0, i)),
        out_shape=jax.ShapeDtypeStruct((_BS, 1, _CV, _HW), f32),
        compiler_params=pltpu.CompilerParams(
            dimension_semantics=("parallel", "parallel"),
        ),
    )(lhs, rhs, v16, pf, lpf, lmv, sens, lm, obj,
      wu1lpf, wu1pf, wu1lm, wu1d, bu1, W_u2, bu2,
      wfpf, wfvr, wfs, wflm, bf_, Wq, Wk, Wv, Wo)
    return out.reshape(_BS, 1, _CV, _H, _W)
